# baseline (speedup 1.0000x reference)
import functools

import jax
import jax.numpy as jnp
from jax import lax
from jax.experimental import pallas as pl
from jax.experimental.pallas import tpu as pltpu
from jax.experimental.pallas import tpu_sc as plsc

N = 4096
M = 4096
E = 64

_SC_INFO = plsc.get_sparse_core_info()
_NC = _SC_INFO.num_cores
_NS = _SC_INFO.num_subcores
_NW = _NC * _NS
_TR = 16


def _make_sc_kernel(b):
    rows_w = b // _NW
    gath_w = rows_w * E
    n_blk = rows_w // _TR
    mesh = plsc.VectorSubcoreMesh(core_axis_name="c", subcore_axis_name="s")

    @functools.partial(
        pl.kernel,
        mesh=mesh,
        out_type=jax.ShapeDtypeStruct((b * M,), jnp.float32),
        scratch_types=[
            pltpu.VMEM((gath_w,), jnp.int32),
            pltpu.VMEM((gath_w,), jnp.int32),
            pltpu.VMEM((gath_w,), jnp.float32),
            pltpu.VMEM((_TR * M,), jnp.float32),
            pltpu.VMEM((E,), jnp.float32),
            pltpu.SemaphoreType.DMA,
            pltpu.SemaphoreType.DMA,
            pltpu.SemaphoreType.DMA,
        ],
    )
    def sck(x_hbm, gidx_hbm, sidx_hbm, vals_hbm, bias_hbm, out_hbm,
            gidx_v, sidx_v, xg_v, tpl_v, vals_v, gsem, rsem, ssem):
        wid = lax.axis_index("s") * _NC + lax.axis_index("c")
        row0 = wid * rows_w
        gbase = pl.multiple_of(row0 * E, 8)

        pltpu.sync_copy(gidx_hbm.at[pl.ds(gbase, gath_w)], gidx_v)
        pltpu.sync_copy(sidx_hbm.at[pl.ds(gbase, gath_w)], sidx_v)
        pltpu.sync_copy(vals_hbm, vals_v)
        gcopy = pltpu.async_copy(x_hbm.at[gidx_v], xg_v, gsem)
        for t in range(_TR):
            pltpu.sync_copy(bias_hbm, tpl_v.at[pl.ds(t * M, M)])

        def blkfire(g, carry):
            off = pl.multiple_of((row0 + g * _TR) * M, 4096)
            pltpu.async_copy(tpl_v, out_hbm.at[pl.ds(off, _TR * M)], rsem)
            return carry

        lax.fori_loop(0, n_blk, blkfire, 0)

        gcopy.wait()

        def mul(g, carry):
            base = pl.multiple_of(g * 256, 256)
            for u in range(16):
                xg_v[pl.ds(base + 16 * u, 16)] = (
                    xg_v[pl.ds(base + 16 * u, 16)]
                    * vals_v[pl.ds(16 * (u % 4), 16)]
                )
            return carry

        lax.fori_loop(0, gath_w // 256, mul, 0)

        def blkdrain(g, carry):
            pltpu.make_async_copy(
                tpl_v, out_hbm.at[pl.ds(0, _TR * M)], rsem
            ).wait()
            return carry

        lax.fori_loop(0, n_blk, blkdrain, 0)

        pltpu.async_copy(xg_v, out_hbm.at[sidx_v], ssem).wait()

    return sck


@jax.jit
def kernel(x, indices, values, bias):
    b = x.shape[0]
    xflat = x.reshape(b * N)
    offs = jnp.arange(b, dtype=jnp.int32)[:, None]
    gidx = (offs * N + indices[0][None, :]).reshape(b * E)
    sidx = (offs * M + indices[1][None, :]).reshape(b * E)
    out = _make_sc_kernel(b)(xflat, gidx, sidx, values, bias.reshape(M))
    return out.reshape(b, M, 1)


# speedup vs baseline: 1.0037x; 1.0037x over previous
import functools

import jax
import jax.numpy as jnp
from jax import lax
from jax.experimental import pallas as pl
from jax.experimental.pallas import tpu as pltpu
from jax.experimental.pallas import tpu_sc as plsc

N = 4096
M = 4096
E = 64

_SC_INFO = plsc.get_sparse_core_info()
_NC = _SC_INFO.num_cores
_NS = _SC_INFO.num_subcores
_NW = _NC * _NS
_TR = 16


def _make_sc_kernel(b):
    rows_w = b // _NW
    gath_w = rows_w * E
    n_blk = rows_w // _TR
    mesh = plsc.VectorSubcoreMesh(core_axis_name="c", subcore_axis_name="s")

    @functools.partial(
        pl.kernel,
        mesh=mesh,
        out_type=jax.ShapeDtypeStruct((b * M,), jnp.float32),
        scratch_types=[
            pltpu.VMEM((gath_w,), jnp.int32),
            pltpu.VMEM((gath_w,), jnp.int32),
            pltpu.VMEM((gath_w,), jnp.float32),
            pltpu.VMEM((_TR * M,), jnp.float32),
            pltpu.VMEM((E,), jnp.float32),
            pltpu.SemaphoreType.DMA,
            pltpu.SemaphoreType.DMA,
            pltpu.SemaphoreType.DMA,
        ],
    )
    def sck(x_hbm, gidx_hbm, sidx_hbm, vals_hbm, bias_hbm, out_hbm,
            gidx_v, sidx_v, xg_v, tpl_v, vals_v, gsem, rsem, ssem):
        wid = lax.axis_index("s") * _NC + lax.axis_index("c")
        row0 = wid * rows_w
        gbase = pl.multiple_of(row0 * E, 8)

        with jax.named_scope("ph_stage"):
            pltpu.sync_copy(gidx_hbm.at[pl.ds(gbase, gath_w)], gidx_v)
            pltpu.sync_copy(sidx_hbm.at[pl.ds(gbase, gath_w)], sidx_v)
            pltpu.sync_copy(vals_hbm, vals_v)
            gcopy = pltpu.async_copy(x_hbm.at[gidx_v], xg_v, gsem)
            for t in range(_TR):
                pltpu.sync_copy(bias_hbm, tpl_v.at[pl.ds(t * M, M)])

        def blkfire(g, carry):
            off = pl.multiple_of((row0 + g * _TR) * M, 4096)
            pltpu.async_copy(tpl_v, out_hbm.at[pl.ds(off, _TR * M)], rsem)
            return carry

        with jax.named_scope("ph_blkfire"):
            lax.fori_loop(0, n_blk, blkfire, 0)

        with jax.named_scope("ph_gwait"):
            gcopy.wait()

        def mul(g, carry):
            base = pl.multiple_of(g * 256, 256)
            for u in range(16):
                xg_v[pl.ds(base + 16 * u, 16)] = (
                    xg_v[pl.ds(base + 16 * u, 16)]
                    * vals_v[pl.ds(16 * (u % 4), 16)]
                )
            return carry

        with jax.named_scope("ph_mul"):
            lax.fori_loop(0, gath_w // 256, mul, 0)

        def blkdrain(g, carry):
            pltpu.make_async_copy(
                tpl_v, out_hbm.at[pl.ds(0, _TR * M)], rsem
            ).wait()
            return carry

        with jax.named_scope("ph_drain"):
            lax.fori_loop(0, n_blk, blkdrain, 0)

        with jax.named_scope("ph_scatter"):
            pltpu.async_copy(xg_v, out_hbm.at[sidx_v], ssem).wait()

    return sck


@jax.jit
def kernel(x, indices, values, bias):
    b = x.shape[0]
    xflat = x.reshape(b * N)
    offs = jnp.arange(b, dtype=jnp.int32)[:, None]
    gidx = (offs * N + indices[0][None, :]).reshape(b * E)
    sidx = (offs * M + indices[1][None, :]).reshape(b * E)
    out = _make_sc_kernel(b)(xflat, gidx, sidx, values, bias.reshape(M))
    return out.reshape(b, M, 1)


# speedup vs baseline: 6.4460x; 6.4220x over previous
import functools

import jax
import jax.numpy as jnp
from jax import lax
from jax.experimental import pallas as pl
from jax.experimental.pallas import tpu as pltpu
from jax.experimental.pallas import tpu_sc as plsc

N = 4096
M = 4096
E = 64
BB = 256
_C = M // 128

_SC_INFO = plsc.get_sparse_core_info()
_NC = _SC_INFO.num_cores
_NS = _SC_INFO.num_subcores
_NW = _NC * _NS


def _make_sc_gather(b):
    rows_w = b // _NW
    per_w = rows_w * E
    mesh = plsc.VectorSubcoreMesh(core_axis_name="c", subcore_axis_name="s")

    @functools.partial(
        pl.kernel,
        mesh=mesh,
        out_type=jax.ShapeDtypeStruct((b * E,), jnp.float32),
        scratch_types=[
            pltpu.VMEM((E,), jnp.int32),
            pltpu.VMEM((per_w,), jnp.int32),
            pltpu.VMEM((per_w,), jnp.float32),
            pltpu.SemaphoreType.DMA,
        ],
    )
    def gather_k(xflat_hbm, src_hbm, out_hbm, src_v, idx_v, val_v, sem):
        wid = lax.axis_index("s") * _NC + lax.axis_index("c")
        base = pl.multiple_of(wid * per_w, 8)
        row0 = wid * rows_w
        pltpu.sync_copy(src_hbm, src_v)

        def mkidx(r, carry):
            rowoff = (row0 + r) * N
            rb = pl.multiple_of(r * E, 16)
            for j in range(E // 16):
                idx_v[pl.ds(rb + 16 * j, 16)] = (
                    src_v[pl.ds(16 * j, 16)] + rowoff
                )
            return carry

        lax.fori_loop(0, rows_w, mkidx, 0)
        pltpu.async_copy(xflat_hbm.at[idx_v], val_v, sem).wait()
        pltpu.sync_copy(val_v, out_hbm.at[pl.ds(base, per_w)])

    return gather_k


def _tile_body(idx_ref, dstcol_ref, valscol_ref, bias_ref, xg_ref, out_ref):
    dst_row = idx_ref[1:2, :]
    dstcol = dstcol_ref[...]
    valscol = valscol_ref[...]

    j_iota = jax.lax.broadcasted_iota(jnp.int32, (E, 128), 1)
    s_mat = jnp.where(j_iota == dstcol % 128, valscol, 0.0)

    c_iota = jax.lax.broadcasted_iota(jnp.int32, (_C, E), 0)
    mask = (c_iota == dst_row // 128).astype(jnp.float32)

    xg = xg_ref[...]
    xg3 = (xg[:, None, :] * mask[None, :, :]).reshape(BB * _C, E)
    part = jax.lax.dot_general(
        xg3, s_mat,
        dimension_numbers=(((1,), (0,)), ((), ())),
        preferred_element_type=jnp.float32,
    )

    bias_blk = jnp.broadcast_to(bias_ref[...][None], (BB, _C, 128))
    out_ref[...] = part + bias_blk.reshape(BB * _C, 128)


@jax.jit
def kernel(x, indices, values, bias):
    b = x.shape[0]
    xflat = x.reshape(b * N)
    xg = _make_sc_gather(b)(xflat, indices[0]).reshape(b, E)

    dstcol = indices[1].reshape(E, 1)
    valscol = values.reshape(E, 1)
    bias32 = bias.reshape(_C, 128)
    out = pl.pallas_call(
        _tile_body,
        grid=(b // BB,),
        in_specs=[
            pl.BlockSpec((2, E), lambda i: (0, 0)),
            pl.BlockSpec((E, 1), lambda i: (0, 0)),
            pl.BlockSpec((E, 1), lambda i: (0, 0)),
            pl.BlockSpec((_C, 128), lambda i: (0, 0)),
            pl.BlockSpec((BB, E), lambda i: (i, 0)),
        ],
        out_specs=pl.BlockSpec((BB * _C, 128), lambda i: (i, 0)),
        out_shape=jax.ShapeDtypeStruct((b * _C, 128), jnp.float32),
    )(indices, dstcol, valscol, bias32, xg)
    return out.reshape(b, M, 1)


# speedup vs baseline: 7.1511x; 1.1094x over previous
import functools

import jax
import jax.numpy as jnp
from jax import lax
from jax.experimental import pallas as pl
from jax.experimental.pallas import tpu as pltpu
from jax.experimental.pallas import tpu_sc as plsc

N = 4096
M = 4096
E = 64
BB = 256
_C = M // 128
_XR = BB // 2

_SC_INFO = plsc.get_sparse_core_info()
_NC = _SC_INFO.num_cores
_NS = _SC_INFO.num_subcores
_NW = _NC * _NS


def _make_sc_gather(total):
    per_w = total // _NW
    mesh = plsc.VectorSubcoreMesh(core_axis_name="c", subcore_axis_name="s")

    @functools.partial(
        pl.kernel,
        mesh=mesh,
        out_type=jax.ShapeDtypeStruct((total,), jnp.float32),
        scratch_types=[
            pltpu.VMEM((per_w,), jnp.int32),
            pltpu.VMEM((per_w,), jnp.float32),
            pltpu.SemaphoreType.DMA,
        ],
    )
    def gather_k(xflat_hbm, idx_hbm, out_hbm, idx_v, val_v, sem):
        wid = lax.axis_index("s") * _NC + lax.axis_index("c")
        base = pl.multiple_of(wid * per_w, 8)
        pltpu.sync_copy(idx_hbm.at[pl.ds(base, per_w)], idx_v)
        pltpu.async_copy(xflat_hbm.at[idx_v], val_v, sem).wait()
        pltpu.sync_copy(val_v, out_hbm.at[pl.ds(base, per_w)])

    return gather_k


def _tile_body(dstdup_ref, dstcol_ref, valscol_ref, bias_ref, xg_ref, out_ref):
    dstdup = dstdup_ref[...]
    dstcol = dstcol_ref[...]
    valscol = valscol_ref[...]

    j_iota = jax.lax.broadcasted_iota(jnp.int32, (2 * E, 128), 1)
    s_mat = jnp.where(j_iota == dstcol % 128, valscol, 0.0)

    q_iota = jax.lax.broadcasted_iota(jnp.int32, (2 * _C, 2 * E), 0)
    f_iota = jax.lax.broadcasted_iota(jnp.int32, (2 * _C, 2 * E), 1)
    bigmask = (
        (f_iota // E == q_iota // _C) & (dstdup // 128 == q_iota % _C)
    ).astype(jnp.float32)

    xg2 = xg_ref[...]
    xg4 = (xg2[:, None, :] * bigmask[None, :, :]).reshape(_XR * 2 * _C, 2 * E)
    part = jax.lax.dot_general(
        xg4, s_mat,
        dimension_numbers=(((1,), (0,)), ((), ())),
        preferred_element_type=jnp.float32,
    )

    bias2 = jnp.concatenate([bias_ref[...], bias_ref[...]], axis=0)
    bias_blk = jnp.broadcast_to(bias2[None], (_XR, 2 * _C, 128))
    out_ref[...] = part + bias_blk.reshape(BB * _C, 128)


@jax.jit
def kernel(x, indices, values, bias):
    b = x.shape[0]
    xflat = x.reshape(b * N)
    flat_idx = (
        jnp.arange(b, dtype=jnp.int32)[:, None] * N + indices[0][None, :]
    ).reshape(b * E)
    xg2d = _make_sc_gather(b * E)(xflat, flat_idx).reshape(b * E // 128, 128)

    dst2 = jnp.tile(indices[1], 2)
    dstdup = dst2.reshape(1, 2 * E)
    dstcol = dst2.reshape(2 * E, 1)
    valscol = jnp.tile(values, 2).reshape(2 * E, 1)
    bias32 = bias.reshape(_C, 128)
    out = pl.pallas_call(
        _tile_body,
        grid=(b // BB,),
        in_specs=[
            pl.BlockSpec((1, 2 * E), lambda i: (0, 0)),
            pl.BlockSpec((2 * E, 1), lambda i: (0, 0)),
            pl.BlockSpec((2 * E, 1), lambda i: (0, 0)),
            pl.BlockSpec((_C, 128), lambda i: (0, 0)),
            pl.BlockSpec((_XR, 128), lambda i: (i, 0)),
        ],
        out_specs=pl.BlockSpec((BB * _C, 128), lambda i: (i, 0)),
        out_shape=jax.ShapeDtypeStruct((b * _C, 128), jnp.float32),
    )(dstdup, dstcol, valscol, bias32, xg2d)
    return out.reshape(b, M, 1)
